# Initial kernel scaffold; baseline (speedup 1.0000x reference)
import jax
import jax.numpy as jnp
from jax.experimental import pallas as pl


def kernel(x, edge_index, W0, W1, b1, W2, b2):
    raise NotImplementedError("write your pallas kernel here")


# speedup vs baseline: 22.6017x; 22.6017x over previous
import functools

import jax
import jax.numpy as jnp
from jax import lax
from jax.experimental import pallas as pl
from jax.experimental.pallas import tpu as pltpu
from jax.experimental.pallas import tpu_sc as plsc

N_NODES = 10000
N_EDGES = 320000
D_IN = 128
D_HID = 128
D_OUT = 64

NPAD = 10240
EPAD = 327680
N_WORKERS = 32
CHUNK = 128
CHUNKS_PER_W = EPAD // (N_WORKERS * CHUNK)
ROWS_PER_TILE = NPAD // 16

_sc_mesh = plsc.VectorSubcoreMesh(core_axis_name="c", subcore_axis_name="s")


def _deg_body(dst_hbm, zeros_hbm, out_hbm, acc, idxv, ones, sem):
    c = lax.axis_index("c")
    s = lax.axis_index("s")
    wid = c * 16 + s
    base = s * ROWS_PER_TILE
    pltpu.sync_copy(zeros_hbm.at[pl.ds(base, ROWS_PER_TILE)],
                    acc.at[pl.ds(base, ROWS_PER_TILE)])
    for i in range(CHUNK // 16):
        ones[pl.ds(i * 16, 16)] = jnp.full((16,), 1.0, jnp.float32)
    pltpu.sync_copy(dst_hbm.at[pl.ds(wid * CHUNKS_PER_W, CHUNKS_PER_W)], idxv)
    plsc.subcore_barrier()

    def chunk(j, carry):
        pltpu.sync_copy(ones, acc.at[idxv.at[j]], add=True)
        return carry

    lax.fori_loop(0, CHUNKS_PER_W, chunk, 0)
    plsc.subcore_barrier()
    pltpu.sync_copy(acc.at[pl.ds(base, ROWS_PER_TILE)],
                    out_hbm.at[c, pl.ds(base, ROWS_PER_TILE)])


_deg_kernel = pl.kernel(
    _deg_body,
    out_type=jax.ShapeDtypeStruct((2, NPAD), jnp.float32),
    mesh=_sc_mesh,
    scratch_types=[
        pltpu.VMEM_SHARED((NPAD,), jnp.float32),
        pltpu.VMEM((CHUNKS_PER_W, CHUNK), jnp.int32),
        pltpu.VMEM((CHUNK,), jnp.float32),
        pltpu.SemaphoreType.DMA,
    ],
)


def _scat_body(table_hbm, src_hbm, dst_hbm, zeros_hbm, out_hbm,
               acc, srcv, dstv, rows, sem):
    c = lax.axis_index("c")
    s = lax.axis_index("s")
    wid = c * 16 + s
    base = s * ROWS_PER_TILE
    pltpu.sync_copy(zeros_hbm.at[pl.ds(base, ROWS_PER_TILE)],
                    acc.at[pl.ds(base, ROWS_PER_TILE)])
    pltpu.sync_copy(src_hbm.at[pl.ds(wid * CHUNKS_PER_W, CHUNKS_PER_W)], srcv)
    pltpu.sync_copy(dst_hbm.at[pl.ds(wid * CHUNKS_PER_W, CHUNKS_PER_W)], dstv)
    plsc.subcore_barrier()

    def chunk(j, carry):
        pltpu.async_copy(table_hbm.at[srcv.at[j]], rows, sem).wait()
        pltpu.sync_copy(rows, acc.at[dstv.at[j]], add=True)
        return carry

    lax.fori_loop(0, CHUNKS_PER_W, chunk, 0)
    plsc.subcore_barrier()
    pltpu.sync_copy(acc.at[pl.ds(base, ROWS_PER_TILE)],
                    out_hbm.at[c, pl.ds(base, ROWS_PER_TILE)])


def _make_scat_kernel(d):
    return pl.kernel(
        _scat_body,
        out_type=jax.ShapeDtypeStruct((2, NPAD, d), jnp.float32),
        mesh=_sc_mesh,
        compiler_params=pltpu.CompilerParams(use_tc_tiling_on_sc=False),
        scratch_types=[
            pltpu.VMEM_SHARED((NPAD, d), jnp.float32),
            pltpu.VMEM((CHUNKS_PER_W, CHUNK), jnp.int32),
            pltpu.VMEM((CHUNKS_PER_W, CHUNK), jnp.int32),
            pltpu.VMEM((CHUNK, d), jnp.float32),
            pltpu.SemaphoreType.DMA,
        ],
    )


_scat_hid = _make_scat_kernel(D_HID)
_scat_out = _make_scat_kernel(D_OUT)


def _w01_body(w0_ref, w1_ref, out_ref):
    out_ref[...] = jnp.dot(w0_ref[...], w1_ref[...],
                           preferred_element_type=jnp.float32)


def _tc1_body(x_ref, w_ref, d0_ref, d1_ref, g1_ref, dinv_ref):
    d = d0_ref[...] + d1_ref[...] + 1.0
    dinv = lax.rsqrt(d)
    h = jnp.dot(x_ref[...], w_ref[...], preferred_element_type=jnp.float32)
    g1_ref[...] = h * dinv
    dinv_ref[...] = dinv


def _tc2_body(p0_ref, p1_ref, g1_ref, dinv_ref, b1_ref, w2_ref, g2_ref):
    dinv = dinv_ref[...]
    agg = (p0_ref[...] + p1_ref[...] + g1_ref[...]) * dinv + b1_ref[...]
    h = jnp.maximum(agg, 0.0)
    g2_ref[...] = jnp.dot(h, w2_ref[...],
                          preferred_element_type=jnp.float32) * dinv


def _tc3_body(q0_ref, q1_ref, g2_ref, dinv_ref, b2_ref, out_ref):
    out_ref[...] = ((q0_ref[...] + q1_ref[...] + g2_ref[...]) * dinv_ref[...]
                    + b2_ref[...])


_BLK = 512
_NBLK = NPAD // _BLK


def _row_spec(d):
    return pl.BlockSpec((_BLK, d), lambda i: (i, 0))


def _full_spec(r, c):
    return pl.BlockSpec((r, c), lambda i: (0, 0))


_w01_call = pl.pallas_call(
    _w01_body, out_shape=jax.ShapeDtypeStruct((D_IN, D_HID), jnp.float32))

_tc1_call = pl.pallas_call(
    _tc1_body,
    grid=(_NBLK,),
    in_specs=[_row_spec(D_IN), _full_spec(D_IN, D_HID),
              _row_spec(1), _row_spec(1)],
    out_specs=[_row_spec(D_HID), _row_spec(1)],
    out_shape=[jax.ShapeDtypeStruct((NPAD, D_HID), jnp.float32),
               jax.ShapeDtypeStruct((NPAD, 1), jnp.float32)],
)

_tc2_call = pl.pallas_call(
    _tc2_body,
    grid=(_NBLK,),
    in_specs=[_row_spec(D_HID), _row_spec(D_HID), _row_spec(D_HID),
              _row_spec(1), _full_spec(1, D_HID), _full_spec(D_HID, D_OUT)],
    out_specs=_row_spec(D_OUT),
    out_shape=jax.ShapeDtypeStruct((NPAD, D_OUT), jnp.float32),
)

_tc3_call = pl.pallas_call(
    _tc3_body,
    grid=(_NBLK,),
    in_specs=[_row_spec(D_OUT), _row_spec(D_OUT), _row_spec(D_OUT),
              _row_spec(1), _full_spec(1, D_OUT)],
    out_specs=_row_spec(D_OUT),
    out_shape=jax.ShapeDtypeStruct((NPAD, D_OUT), jnp.float32),
)


@jax.jit
def kernel(x, edge_index, W0, W1, b1, W2, b2):
    npadrows = NPAD - N_NODES
    pad_idx = (N_NODES
               + jnp.arange(EPAD - N_EDGES, dtype=jnp.int32) % npadrows)
    src = jnp.concatenate([edge_index[0], pad_idx]).reshape(-1, CHUNK)
    dst = jnp.concatenate([edge_index[1], pad_idx]).reshape(-1, CHUNK)
    x_pad = jnp.zeros((NPAD, D_IN), x.dtype).at[:N_NODES].set(x)
    zeros1 = jnp.zeros((NPAD,), jnp.float32)
    zeros_h = jnp.zeros((NPAD, D_HID), jnp.float32)
    zeros_o = jnp.zeros((NPAD, D_OUT), jnp.float32)

    degp = _deg_kernel(dst, zeros1)
    d0 = degp[0][:, None]
    d1 = degp[1][:, None]

    w01 = _w01_call(W0, W1)
    g1, dinv = _tc1_call(x_pad, w01, d0, d1)

    p = _scat_hid(g1, src, dst, zeros_h)

    g2 = _tc2_call(p[0], p[1], g1, dinv, b1[None, :], W2)

    q = _scat_out(g2, src, dst, zeros_o)

    out = _tc3_call(q[0], q[1], g2, dinv, b2[None, :])
    return out[:N_NODES]


# speedup vs baseline: 29.5051x; 1.3054x over previous
import functools

import jax
import jax.numpy as jnp
from jax import lax
from jax.experimental import pallas as pl
from jax.experimental.pallas import tpu as pltpu
from jax.experimental.pallas import tpu_sc as plsc

N_NODES = 10000
N_EDGES = 320000
D_IN = 128
D_HID = 128
D_OUT = 64

NPAD = 10240
EPAD = 327680
N_WORKERS = 32
CHUNK = 128
CHUNKS_PER_W = EPAD // (N_WORKERS * CHUNK)
ROWS_PER_TILE = NPAD // 16

_sc_mesh = plsc.VectorSubcoreMesh(core_axis_name="c", subcore_axis_name="s")


_DEG_GRP = 8


def _deg_body(dst_hbm, zeros_hbm, out_hbm, acc, idxv, ones, sem):
    c = lax.axis_index("c")
    s = lax.axis_index("s")
    wid = c * 16 + s
    base = s * ROWS_PER_TILE
    pltpu.sync_copy(zeros_hbm.at[pl.ds(base, ROWS_PER_TILE)],
                    acc.at[pl.ds(base, ROWS_PER_TILE)])
    for i in range(CHUNK // 16):
        ones[pl.ds(i * 16, 16)] = jnp.full((16,), 1.0, jnp.float32)
    pltpu.sync_copy(dst_hbm.at[pl.ds(wid * CHUNKS_PER_W, CHUNKS_PER_W)], idxv)
    plsc.subcore_barrier()

    def group(g, carry):
        for j in range(_DEG_GRP):
            pltpu.async_copy(ones, acc.at[idxv.at[g * _DEG_GRP + j]], sem,
                             add=True)
        for j in range(_DEG_GRP):
            pltpu.make_async_copy(ones, acc.at[idxv.at[g * _DEG_GRP + j]],
                                  sem).wait()
        return carry

    lax.fori_loop(0, CHUNKS_PER_W // _DEG_GRP, group, 0)
    plsc.subcore_barrier()
    pltpu.sync_copy(acc.at[pl.ds(base, ROWS_PER_TILE)],
                    out_hbm.at[c, pl.ds(base, ROWS_PER_TILE)])


_deg_kernel = pl.kernel(
    _deg_body,
    out_type=jax.ShapeDtypeStruct((2, NPAD), jnp.float32),
    mesh=_sc_mesh,
    scratch_types=[
        pltpu.VMEM_SHARED((NPAD,), jnp.float32),
        pltpu.VMEM((CHUNKS_PER_W, CHUNK), jnp.int32),
        pltpu.VMEM((CHUNK,), jnp.float32),
        pltpu.SemaphoreType.DMA,
    ],
)


_GRP = 16
_NGRP = CHUNKS_PER_W // _GRP


def _scat_body(table_hbm, src_hbm, dst_hbm, zeros_hbm, out_hbm,
               acc, srcv, dstv, rows0, rows1, sem0, sem1):
    c = lax.axis_index("c")
    s = lax.axis_index("s")
    wid = c * 16 + s
    base = s * ROWS_PER_TILE
    pltpu.sync_copy(zeros_hbm.at[pl.ds(base, ROWS_PER_TILE)],
                    acc.at[pl.ds(base, ROWS_PER_TILE)])
    plsc.subcore_barrier()

    def gat(j, rows, sem):
        pltpu.async_copy(table_hbm.at[srcv.at[j]], rows, sem)

    def wait_gat(j, rows, sem):
        pltpu.make_async_copy(table_hbm.at[srcv.at[j]], rows, sem).wait()

    def group(g, carry):
        grow = wid * CHUNKS_PER_W + g * _GRP
        pltpu.sync_copy(src_hbm.at[pl.ds(grow, _GRP)], srcv)
        pltpu.sync_copy(dst_hbm.at[pl.ds(grow, _GRP)], dstv)
        gat(0, rows0, sem0)

        def pair(k, carry2):
            j0 = 2 * k
            gat(j0 + 1, rows1, sem1)
            wait_gat(j0, rows0, sem0)
            pltpu.sync_copy(rows0, acc.at[dstv.at[j0]], add=True)
            gat(j0 + 2, rows0, sem0)
            wait_gat(j0 + 1, rows1, sem1)
            pltpu.sync_copy(rows1, acc.at[dstv.at[j0 + 1]], add=True)
            return carry2

        lax.fori_loop(0, _GRP // 2 - 1, pair, 0)
        j0 = _GRP - 2
        gat(j0 + 1, rows1, sem1)
        wait_gat(j0, rows0, sem0)
        pltpu.sync_copy(rows0, acc.at[dstv.at[j0]], add=True)
        wait_gat(j0 + 1, rows1, sem1)
        pltpu.sync_copy(rows1, acc.at[dstv.at[j0 + 1]], add=True)
        return carry

    lax.fori_loop(0, _NGRP, group, 0)
    plsc.subcore_barrier()
    pltpu.sync_copy(acc.at[pl.ds(base, ROWS_PER_TILE)],
                    out_hbm.at[c, pl.ds(base, ROWS_PER_TILE)])


def _make_scat_kernel(d):
    return pl.kernel(
        _scat_body,
        out_type=jax.ShapeDtypeStruct((2, NPAD, d), jnp.float32),
        mesh=_sc_mesh,
        compiler_params=pltpu.CompilerParams(use_tc_tiling_on_sc=False),
        scratch_types=[
            pltpu.VMEM_SHARED((NPAD, d), jnp.float32),
            pltpu.VMEM((_GRP, CHUNK), jnp.int32),
            pltpu.VMEM((_GRP, CHUNK), jnp.int32),
            pltpu.VMEM((CHUNK, d), jnp.float32),
            pltpu.VMEM((CHUNK, d), jnp.float32),
            pltpu.SemaphoreType.DMA,
            pltpu.SemaphoreType.DMA,
        ],
    )


_scat_hid = _make_scat_kernel(D_HID)
_scat_out = _make_scat_kernel(D_OUT)


def _tc1_body(x_ref, w0_ref, w1_ref, d0_ref, d1_ref, g1_ref, dinv_ref):
    d = d0_ref[...] + d1_ref[...] + 1.0
    dinv = lax.rsqrt(d)
    w01 = jnp.dot(w0_ref[...], w1_ref[...], preferred_element_type=jnp.float32)
    h = jnp.dot(x_ref[...], w01, preferred_element_type=jnp.float32)
    g1_ref[...] = h * dinv
    dinv_ref[...] = dinv


def _tc2_body(p0_ref, p1_ref, g1_ref, dinv_ref, b1_ref, w2_ref, g2_ref):
    dinv = dinv_ref[...]
    agg = (p0_ref[...] + p1_ref[...] + g1_ref[...]) * dinv + b1_ref[...]
    h = jnp.maximum(agg, 0.0)
    g2_ref[...] = jnp.dot(h, w2_ref[...],
                          preferred_element_type=jnp.float32) * dinv


def _tc3_body(q0_ref, q1_ref, g2_ref, dinv_ref, b2_ref, out_ref):
    out_ref[...] = ((q0_ref[...] + q1_ref[...] + g2_ref[...]) * dinv_ref[...]
                    + b2_ref[...])


_BLK = 512
_NBLK = NPAD // _BLK


def _row_spec(d):
    return pl.BlockSpec((_BLK, d), lambda i: (i, 0))


def _full_spec(r, c):
    return pl.BlockSpec((r, c), lambda i: (0, 0))


_tc1_call = pl.pallas_call(
    _tc1_body,
    grid=(_NBLK,),
    in_specs=[_row_spec(D_IN), _full_spec(D_IN, D_HID), _full_spec(D_HID, D_HID),
              _row_spec(1), _row_spec(1)],
    out_specs=[_row_spec(D_HID), _row_spec(1)],
    out_shape=[jax.ShapeDtypeStruct((NPAD, D_HID), jnp.float32),
               jax.ShapeDtypeStruct((NPAD, 1), jnp.float32)],
)

_tc2_call = pl.pallas_call(
    _tc2_body,
    grid=(_NBLK,),
    in_specs=[_row_spec(D_HID), _row_spec(D_HID), _row_spec(D_HID),
              _row_spec(1), _full_spec(1, D_HID), _full_spec(D_HID, D_OUT)],
    out_specs=_row_spec(D_OUT),
    out_shape=jax.ShapeDtypeStruct((NPAD, D_OUT), jnp.float32),
)

_tc3_call = pl.pallas_call(
    _tc3_body,
    grid=(_NBLK,),
    in_specs=[_row_spec(D_OUT), _row_spec(D_OUT), _row_spec(D_OUT),
              _row_spec(1), _full_spec(1, D_OUT)],
    out_specs=_row_spec(D_OUT),
    out_shape=jax.ShapeDtypeStruct((NPAD, D_OUT), jnp.float32),
)


@jax.jit
def kernel(x, edge_index, W0, W1, b1, W2, b2):
    npadrows = NPAD - N_NODES
    pad_idx = (N_NODES
               + jnp.arange(EPAD - N_EDGES, dtype=jnp.int32) % npadrows)
    src = jnp.concatenate([edge_index[0], pad_idx]).reshape(-1, CHUNK)
    dst = jnp.concatenate([edge_index[1], pad_idx]).reshape(-1, CHUNK)
    x_pad = jnp.zeros((NPAD, D_IN), x.dtype).at[:N_NODES].set(x)
    zeros1 = jnp.zeros((NPAD,), jnp.float32)
    zeros_h = jnp.zeros((NPAD, D_HID), jnp.float32)
    zeros_o = jnp.zeros((NPAD, D_OUT), jnp.float32)

    degp = _deg_kernel(dst, zeros1)
    d0 = degp[0][:, None]
    d1 = degp[1][:, None]

    g1, dinv = _tc1_call(x_pad, W0, W1, d0, d1)

    p = _scat_hid(g1, src, dst, zeros_h)

    g2 = _tc2_call(p[0], p[1], g1, dinv, b1[None, :], W2)

    q = _scat_out(g2, src, dst, zeros_o)

    out = _tc3_call(q[0], q[1], g2, dinv, b2[None, :])
    return out[:N_NODES]
